# baseline (speedup 1.0000x reference)
import functools

import jax
import jax.numpy as jnp
from jax import lax
from jax.experimental import pallas as pl
from jax.experimental.pallas import tpu as pltpu
from jax.experimental.pallas import tpu_sc as plsc

G = 8
NC = 2
NS = 16
CB = 128


CH_NUM = 605
CH_DEN = 1000


def _core_chunks(ch_total):
  ch0 = (ch_total * CH_NUM + CH_DEN - 1) // CH_DEN
  return ch0, ch_total - ch0


def _make_segsum(n_pad, D, ch_total):
  mesh = plsc.VectorSubcoreMesh(
      core_axis_name="c", subcore_axis_name="s", num_cores=NC, num_subcores=NS)
  rows_per_sub = n_pad // NS
  ch0, ch1 = _core_chunks(ch_total)
  ch_max = -(-max(ch0, ch1) // 2) * 2

  @functools.partial(
      pl.kernel,
      out_type=jax.ShapeDtypeStruct((NC, n_pad, D), jnp.float32),
      mesh=mesh,
      scratch_types=[
          pltpu.VMEM((-(-ch_max // 2), CB), jnp.int32),
          pltpu.VMEM((-(-ch_max // 2), CB), jnp.int32),
          pltpu.VMEM((CB, D), jnp.float32),
          pltpu.VMEM((CB, D), jnp.float32),
          pltpu.VMEM_SHARED((n_pad, D), jnp.float32),
          pltpu.SemaphoreType.DMA,
          pltpu.SemaphoreType.DMA,
      ],
  )
  def segsum(x_hbm, zeros_hbm, src_hbm, dst_hbm, out_hbm,
             src_v, dst_v, rows_a, rows_b, acc, gsem, ssem):
    c = lax.axis_index("c")
    s = lax.axis_index("s")
    r0 = s * rows_per_sub
    pltpu.sync_copy(zeros_hbm, acc.at[pl.ds(r0, rows_per_sub)])
    plsc.subcore_barrier()

    half = -(-ch_max // 2)
    n_ch = jnp.where(c == 0, ch0, ch1)
    rows = [rows_a, rows_b]

    for hh in range(2):
      k0 = hh * half
      pltpu.sync_copy(src_hbm.at[c, s, pl.ds(k0, half)], src_v)
      pltpu.sync_copy(dst_hbm.at[c, s, pl.ds(k0, half)], dst_v)
      n_here = jnp.clip(n_ch - k0, 0, half)

      def pair(g, carry, k0=k0):
        descs = []
        scats = []
        for b in range(2):
          k = g * 2 + b
          descs.append(
              pltpu.async_copy(x_hbm.at[src_v.at[k]], rows[b], gsem))
          descs[b].wait()
          scats.append(
              pltpu.async_copy(rows[b], acc.at[dst_v.at[k]], ssem, add=True))
        scats[0].wait()
        scats[1].wait()
        return carry

      lax.fori_loop(0, n_here // 2, pair, 0)

      @pl.when(n_here % 2 == 1)
      def _(k0=k0):
        k = n_here - 1
        pltpu.async_copy(x_hbm.at[src_v.at[k]], rows_a, gsem).wait()
        pltpu.sync_copy(rows_a, acc.at[dst_v.at[k]], add=True)

    plsc.subcore_barrier()
    pltpu.sync_copy(acc.at[pl.ds(r0, rows_per_sub)],
                    out_hbm.at[c, pl.ds(r0, rows_per_sub)])

  return segsum


def _l1_body(p_ref, h_ref, wr_ref, wn_ref, b_ref, o_ref):
  a = p_ref[0] + p_ref[1]
  z = (jnp.dot(h_ref[...], wr_ref[...], preferred_element_type=jnp.float32)
       + jnp.dot(a, wn_ref[...], preferred_element_type=jnp.float32)
       + b_ref[...])
  o_ref[...] = jnp.maximum(z, 0.0)


def _l2_body(p_ref, h1_ref, wr_ref, wn_ref, b_ref, bt_ref,
             h2_ref, sum_ref, cnt_ref):
  i = pl.program_id(0)
  R = h1_ref.shape[0]
  D = h1_ref.shape[1]
  a = p_ref[0] + p_ref[1]
  h2 = (jnp.dot(h1_ref[...], wr_ref[...], preferred_element_type=jnp.float32)
        + jnp.dot(a, wn_ref[...], preferred_element_type=jnp.float32)
        + b_ref[...])
  h2_ref[...] = h2
  gids = lax.broadcasted_iota(jnp.int32, (R, G), 1)
  onehot = (bt_ref[...] == gids).astype(jnp.float32)
  ps = lax.dot_general(onehot, h2, (((0,), (0,)), ((), ())),
                       preferred_element_type=jnp.float32)
  pc = lax.dot_general(onehot, jnp.ones((R, D), jnp.float32),
                       (((0,), (0,)), ((), ())),
                       preferred_element_type=jnp.float32)

  @pl.when(i == 0)
  def _():
    sum_ref[...] = jnp.zeros_like(sum_ref)
    cnt_ref[...] = jnp.zeros_like(cnt_ref)

  sum_ref[...] += ps
  cnt_ref[...] += pc


def _rf_body(h2_ref, bt_ref, sum_ref, cnt_ref, o_ref):
  R = h2_ref.shape[0]
  mean = sum_ref[...] / jnp.maximum(cnt_ref[...], 1.0)
  gids = lax.broadcasted_iota(jnp.int32, (R, G), 1)
  onehot = (bt_ref[...] == gids).astype(jnp.float32)
  o_ref[...] = h2_ref[...] + jnp.dot(onehot, mean,
                                     preferred_element_type=jnp.float32)


def kernel(h, group_idx, batch, W1_root, W1_nbr, b1, W3_root, W3_nbr, b3):
  N, D = h.shape
  E = group_idx.shape[1]
  NW = NC * NS
  ch_total = -(-E // (NS * CB))
  ch0, ch1 = _core_chunks(ch_total)
  ch_max = -(-max(ch0, ch1) // 2) * 2
  n_pad = -(-(N + 1) // (NS * 8)) * (NS * 8)
  rows_per_sub = n_pad // NS
  R = 2000
  grid = N // R

  src = group_idx[0]
  dst = group_idx[1]
  def shard(idx):
    idx = jnp.pad(idx, (0, NS * ch_total * CB - E), constant_values=N)
    half0 = idx[:NS * ch0 * CB].reshape(NS, ch0, CB)
    half1 = idx[NS * ch0 * CB:].reshape(NS, ch1, CB)
    half0 = jnp.pad(half0, ((0, 0), (0, ch_max - ch0), (0, 0)),
                    constant_values=N)
    half1 = jnp.pad(half1, ((0, 0), (0, ch_max - ch1), (0, 0)),
                    constant_values=N)
    return jnp.stack([half0, half1])

  src_r = shard(src)
  dst_r = shard(dst)
  h_pad = jnp.pad(h, ((0, n_pad - N), (0, 0)))
  zeros = jnp.zeros((rows_per_sub, D), jnp.float32)
  bt = batch.reshape(N, 1)

  segsum = _make_segsum(n_pad, D, ch_total)

  p1 = segsum(h_pad, zeros, src_r, dst_r)
  h1_pad = pl.pallas_call(
      _l1_body,
      grid=(grid,),
      in_specs=[
          pl.BlockSpec((NC, R, D), lambda i: (0, i, 0)),
          pl.BlockSpec((R, D), lambda i: (i, 0)),
          pl.BlockSpec((D, D), lambda i: (0, 0)),
          pl.BlockSpec((D, D), lambda i: (0, 0)),
          pl.BlockSpec((1, D), lambda i: (0, 0)),
      ],
      out_specs=pl.BlockSpec((R, D), lambda i: (i, 0)),
      out_shape=jax.ShapeDtypeStruct((n_pad, D), jnp.float32),
  )(p1, h, W1_root, W1_nbr, b1.reshape(1, D))

  p2 = segsum(h1_pad, zeros, src_r, dst_r)
  h2, sums, counts = pl.pallas_call(
      _l2_body,
      grid=(grid,),
      in_specs=[
          pl.BlockSpec((NC, R, D), lambda i: (0, i, 0)),
          pl.BlockSpec((R, D), lambda i: (i, 0)),
          pl.BlockSpec((D, D), lambda i: (0, 0)),
          pl.BlockSpec((D, D), lambda i: (0, 0)),
          pl.BlockSpec((1, D), lambda i: (0, 0)),
          pl.BlockSpec((R, 1), lambda i: (i, 0)),
      ],
      out_specs=(
          pl.BlockSpec((R, D), lambda i: (i, 0)),
          pl.BlockSpec((G, D), lambda i: (0, 0)),
          pl.BlockSpec((G, D), lambda i: (0, 0)),
      ),
      out_shape=(
          jax.ShapeDtypeStruct((N, D), jnp.float32),
          jax.ShapeDtypeStruct((G, D), jnp.float32),
          jax.ShapeDtypeStruct((G, D), jnp.float32),
      ),
  )(p2, h1_pad, W3_root, W3_nbr, b3.reshape(1, D), bt)

  out = pl.pallas_call(
      _rf_body,
      grid=(grid,),
      in_specs=[
          pl.BlockSpec((R, D), lambda i: (i, 0)),
          pl.BlockSpec((R, 1), lambda i: (i, 0)),
          pl.BlockSpec((G, D), lambda i: (0, 0)),
          pl.BlockSpec((G, D), lambda i: (0, 0)),
      ],
      out_specs=pl.BlockSpec((R, D), lambda i: (i, 0)),
      out_shape=jax.ShapeDtypeStruct((N, D), jnp.float32),
  )(h2, bt, sums, counts)
  return out


# speedup vs baseline: 1.0594x; 1.0594x over previous
import functools

import jax
import jax.numpy as jnp
from jax import lax
from jax.experimental import pallas as pl
from jax.experimental.pallas import tpu as pltpu
from jax.experimental.pallas import tpu_sc as plsc

G = 8
NC = 2
NS = 16
CB = 128


CH_NUM = 610
CH_DEN = 1000


def _core_chunks(ch_total):
  ch0 = (ch_total * CH_NUM + CH_DEN - 1) // CH_DEN
  return ch0, ch_total - ch0


def _make_segsum(n_pad, D, ch_total):
  mesh = plsc.VectorSubcoreMesh(
      core_axis_name="c", subcore_axis_name="s", num_cores=NC, num_subcores=NS)
  rows_per_sub = n_pad // NS
  ch0, ch1 = _core_chunks(ch_total)
  ch_max = -(-max(ch0, ch1) // 2) * 2

  @functools.partial(
      pl.kernel,
      out_type=jax.ShapeDtypeStruct((NC, n_pad, D), jnp.float32),
      mesh=mesh,
      scratch_types=[
          pltpu.VMEM((-(-ch_max // 2), CB), jnp.int32),
          pltpu.VMEM((-(-ch_max // 2), CB), jnp.int32),
          pltpu.VMEM((CB, D), jnp.float32),
          pltpu.VMEM((CB, D), jnp.float32),
          pltpu.VMEM_SHARED((n_pad, D), jnp.float32),
          pltpu.SemaphoreType.DMA,
          pltpu.SemaphoreType.DMA,
      ],
  )
  def segsum(x_hbm, zeros_hbm, src_hbm, dst_hbm, out_hbm,
             src_v, dst_v, rows_a, rows_b, acc, gsem, ssem):
    c = lax.axis_index("c")
    s = lax.axis_index("s")
    r0 = s * rows_per_sub
    pltpu.sync_copy(zeros_hbm, acc.at[pl.ds(r0, rows_per_sub)])
    plsc.subcore_barrier()

    half = -(-ch_max // 2)
    n_ch = jnp.where(c == 0, ch0, ch1)
    rows = [rows_a, rows_b]

    for hh in range(2):
      k0 = hh * half
      pltpu.sync_copy(src_hbm.at[c, s, pl.ds(k0, half)], src_v)
      pltpu.sync_copy(dst_hbm.at[c, s, pl.ds(k0, half)], dst_v)
      n_here = jnp.clip(n_ch - k0, 0, half)

      def pair(g, carry, k0=k0):
        descs = []
        scats = []
        for b in range(2):
          k = g * 2 + b
          descs.append(
              pltpu.async_copy(x_hbm.at[src_v.at[k]], rows[b], gsem))
          descs[b].wait()
          scats.append(
              pltpu.async_copy(rows[b], acc.at[dst_v.at[k]], ssem, add=True))
        scats[0].wait()
        scats[1].wait()
        return carry

      lax.fori_loop(0, n_here // 2, pair, 0)

      @pl.when(n_here % 2 == 1)
      def _(k0=k0):
        k = n_here - 1
        pltpu.async_copy(x_hbm.at[src_v.at[k]], rows_a, gsem).wait()
        pltpu.sync_copy(rows_a, acc.at[dst_v.at[k]], add=True)

    plsc.subcore_barrier()
    pltpu.sync_copy(acc.at[pl.ds(r0, rows_per_sub)],
                    out_hbm.at[c, pl.ds(r0, rows_per_sub)])

  return segsum


def _l1_body(p_ref, h_ref, wr_ref, wn_ref, b_ref, o_ref):
  a = p_ref[0] + p_ref[1]
  z = (jnp.dot(h_ref[...], wr_ref[...], preferred_element_type=jnp.float32)
       + jnp.dot(a, wn_ref[...], preferred_element_type=jnp.float32)
       + b_ref[...])
  o_ref[...] = jnp.maximum(z, 0.0)


def _l2_body(p_ref, h1_ref, wr_ref, wn_ref, b_ref, bt_ref,
             h2_ref, sum_ref, cnt_ref):
  i = pl.program_id(0)
  R = h1_ref.shape[0]
  D = h1_ref.shape[1]
  a = p_ref[0] + p_ref[1]
  h2 = (jnp.dot(h1_ref[...], wr_ref[...], preferred_element_type=jnp.float32)
        + jnp.dot(a, wn_ref[...], preferred_element_type=jnp.float32)
        + b_ref[...])
  h2_ref[...] = h2
  gids = lax.broadcasted_iota(jnp.int32, (R, G), 1)
  onehot = (bt_ref[...] == gids).astype(jnp.float32)
  ps = lax.dot_general(onehot, h2, (((0,), (0,)), ((), ())),
                       preferred_element_type=jnp.float32)
  pc = lax.dot_general(onehot, jnp.ones((R, D), jnp.float32),
                       (((0,), (0,)), ((), ())),
                       preferred_element_type=jnp.float32)

  @pl.when(i == 0)
  def _():
    sum_ref[...] = jnp.zeros_like(sum_ref)
    cnt_ref[...] = jnp.zeros_like(cnt_ref)

  sum_ref[...] += ps
  cnt_ref[...] += pc


def _rf_body(h2_ref, bt_ref, sum_ref, cnt_ref, o_ref):
  R = h2_ref.shape[0]
  mean = sum_ref[...] / jnp.maximum(cnt_ref[...], 1.0)
  gids = lax.broadcasted_iota(jnp.int32, (R, G), 1)
  onehot = (bt_ref[...] == gids).astype(jnp.float32)
  o_ref[...] = h2_ref[...] + jnp.dot(onehot, mean,
                                     preferred_element_type=jnp.float32)


def kernel(h, group_idx, batch, W1_root, W1_nbr, b1, W3_root, W3_nbr, b3):
  N, D = h.shape
  E = group_idx.shape[1]
  NW = NC * NS
  ch_total = -(-E // (NS * CB))
  ch0, ch1 = _core_chunks(ch_total)
  ch_max = -(-max(ch0, ch1) // 2) * 2
  n_pad = -(-(N + 1) // (NS * 8)) * (NS * 8)
  rows_per_sub = n_pad // NS
  R = 2000
  grid = N // R

  src = group_idx[0]
  dst = group_idx[1]
  def shard(idx):
    idx = jnp.pad(idx, (0, NS * ch_total * CB - E), constant_values=N)
    half0 = idx[:NS * ch0 * CB].reshape(NS, ch0, CB)
    half1 = idx[NS * ch0 * CB:].reshape(NS, ch1, CB)
    half0 = jnp.pad(half0, ((0, 0), (0, ch_max - ch0), (0, 0)),
                    constant_values=N)
    half1 = jnp.pad(half1, ((0, 0), (0, ch_max - ch1), (0, 0)),
                    constant_values=N)
    return jnp.stack([half0, half1])

  src_r = shard(src)
  dst_r = shard(dst)
  h_pad = jnp.pad(h, ((0, n_pad - N), (0, 0)))
  zeros = jnp.zeros((rows_per_sub, D), jnp.float32)
  bt = batch.reshape(N, 1)

  segsum = _make_segsum(n_pad, D, ch_total)

  p1 = segsum(h_pad, zeros, src_r, dst_r)
  h1_pad = pl.pallas_call(
      _l1_body,
      grid=(grid,),
      in_specs=[
          pl.BlockSpec((NC, R, D), lambda i: (0, i, 0)),
          pl.BlockSpec((R, D), lambda i: (i, 0)),
          pl.BlockSpec((D, D), lambda i: (0, 0)),
          pl.BlockSpec((D, D), lambda i: (0, 0)),
          pl.BlockSpec((1, D), lambda i: (0, 0)),
      ],
      out_specs=pl.BlockSpec((R, D), lambda i: (i, 0)),
      out_shape=jax.ShapeDtypeStruct((n_pad, D), jnp.float32),
  )(p1, h, W1_root, W1_nbr, b1.reshape(1, D))

  p2 = segsum(h1_pad, zeros, src_r, dst_r)
  h2, sums, counts = pl.pallas_call(
      _l2_body,
      grid=(grid,),
      in_specs=[
          pl.BlockSpec((NC, R, D), lambda i: (0, i, 0)),
          pl.BlockSpec((R, D), lambda i: (i, 0)),
          pl.BlockSpec((D, D), lambda i: (0, 0)),
          pl.BlockSpec((D, D), lambda i: (0, 0)),
          pl.BlockSpec((1, D), lambda i: (0, 0)),
          pl.BlockSpec((R, 1), lambda i: (i, 0)),
      ],
      out_specs=(
          pl.BlockSpec((R, D), lambda i: (i, 0)),
          pl.BlockSpec((G, D), lambda i: (0, 0)),
          pl.BlockSpec((G, D), lambda i: (0, 0)),
      ),
      out_shape=(
          jax.ShapeDtypeStruct((N, D), jnp.float32),
          jax.ShapeDtypeStruct((G, D), jnp.float32),
          jax.ShapeDtypeStruct((G, D), jnp.float32),
      ),
  )(p2, h1_pad, W3_root, W3_nbr, b3.reshape(1, D), bt)

  out = pl.pallas_call(
      _rf_body,
      grid=(grid,),
      in_specs=[
          pl.BlockSpec((R, D), lambda i: (i, 0)),
          pl.BlockSpec((R, 1), lambda i: (i, 0)),
          pl.BlockSpec((G, D), lambda i: (0, 0)),
          pl.BlockSpec((G, D), lambda i: (0, 0)),
      ],
      out_specs=pl.BlockSpec((R, D), lambda i: (i, 0)),
      out_shape=jax.ShapeDtypeStruct((N, D), jnp.float32),
  )(h2, bt, sums, counts)
  return out
